# baseline (speedup 1.0000x reference)
import functools

import jax
import jax.numpy as jnp
from jax import lax
from jax.experimental import pallas as pl
from jax.experimental.pallas import tpu as pltpu
from jax.experimental.pallas import tpu_sc as plsc

_B, _F, _T = 128, 128, 1024
_NW = 32
_SPW = _B // _NW
_CH = 16
_NCH = _F // _CH
_CHW = _CH * _T
_NBUF = 4
_LANES = 16


def _sc_body(lo_hbm, hi_hbm, x_hbm, o_hbm, lo_v, hi_v, bufs, gsems, ssems):
    cid = lax.axis_index("c")
    sid = lax.axis_index("s")
    wid = cid * 16 + sid

    pltpu.sync_copy(lo_hbm, lo_v.at[pl.ds(0, _B)])
    pltpu.sync_copy(hi_hbm, hi_v.at[pl.ds(0, _B)])
    lo_vec = lo_v[pl.ds(wid * _SPW, _LANES)]
    hi_vec = hi_v[pl.ds(wid * _SPW, _LANES)]

    zvec = jnp.zeros((_LANES,), jnp.float32)

    def gather(k):
        b = wid * _SPW + k // _NCH
        c = k % _NCH
        slot = k % _NBUF
        return pltpu.make_async_copy(
            x_hbm.at[b, pl.ds(c * _CHW, _CHW)], bufs.at[slot], gsems.at[slot]
        )

    def scatter(k):
        b = wid * _SPW + k // _NCH
        c = k % _NCH
        slot = k % _NBUF
        return pltpu.make_async_copy(
            bufs.at[slot], o_hbm.at[b, pl.ds(c * _CHW, _CHW)], ssems.at[slot]
        )

    nk = _SPW * _NCH
    for k in range(min(_NBUF - 1, nk)):
        gather(k).start()

    lo = hi = None
    for k in range(nk):
        c = k % _NCH
        if c == 0:
            lo = lo_vec[k // _NCH]
            hi = hi_vec[k // _NCH]
        gather(k).wait()
        c0 = c * _CH
        s = jnp.clip(lo, c0, c0 + _CH) - c0
        e = jnp.clip(hi, c0, c0 + _CH) - c0
        slot = k % _NBUF

        def zero_seg(j, _, slot=slot):
            bufs[slot, pl.ds(j * _LANES, _LANES)] = zvec
            return 0

        lax.fori_loop(s * (_T // _LANES), e * (_T // _LANES), zero_seg, 0)
        scatter(k).start()
        if k >= 1:
            scatter(k - 1).wait()
        if k + _NBUF - 1 < nk:
            gather(k + _NBUF - 1).start()
    scatter(nk - 1).wait()


def kernel(x):
    mask_ratio = 16
    xs = jnp.squeeze(x, axis=1)
    B, F, T = xs.shape
    max_mask = F // mask_ratio
    k = jax.random.key(42)
    k1, k2 = jax.random.split(k)
    if max_mask == 1:
        f_widths = jnp.ones((B,), dtype=jnp.int32)
    else:
        f_widths = jax.random.randint(k1, (B,), 1, max_mask).astype(jnp.int32)
    u = jax.random.uniform(k2, (B,))
    f_low = jnp.floor(u * (F - f_widths).astype(jnp.float32)).astype(jnp.int32)
    f_hi = f_low + f_widths

    xs2 = xs.reshape(B, F * T)
    run = functools.partial(
        pl.kernel,
        out_type=jax.ShapeDtypeStruct((B, F * T), jnp.float32),
        mesh=plsc.VectorSubcoreMesh(core_axis_name="c", subcore_axis_name="s"),
        scratch_types=[
            pltpu.VMEM((_B + _LANES,), jnp.int32),
            pltpu.VMEM((_B + _LANES,), jnp.int32),
            pltpu.VMEM((_NBUF, _CHW), jnp.float32),
            pltpu.SemaphoreType.DMA((_NBUF,)),
            pltpu.SemaphoreType.DMA((_NBUF,)),
        ],
    )(_sc_body)
    out = run(f_low, f_hi, xs2)
    return out.reshape(B, F, T)[:, None, :, :]


# speedup vs baseline: 1.9751x; 1.9751x over previous
import functools

import jax
import jax.numpy as jnp
from jax import lax
from jax.experimental import pallas as pl
from jax.experimental.pallas import tpu as pltpu
from jax.experimental.pallas import tpu_sc as plsc

_B, _F, _T = 128, 128, 1024
_NW = 32
_SPW = _B // _NW
_CH = 16
_NCH = _F // _CH
_NBUF = 4
_LANES = 16


def _sc_body(lo_hbm, hi_hbm, x_hbm, o_hbm, lo_v, hi_v, b0, b1, b2, b3,
             gsems, ssems):
    cid = lax.axis_index("c")
    sid = lax.axis_index("s")
    wid = cid * 16 + sid
    bufs = (b0, b1, b2, b3)

    pltpu.sync_copy(lo_hbm, lo_v.at[pl.ds(0, _B)])
    pltpu.sync_copy(hi_hbm, hi_v.at[pl.ds(0, _B)])
    lo_vec = lo_v[pl.ds(wid * _SPW, _LANES)]
    hi_vec = hi_v[pl.ds(wid * _SPW, _LANES)]

    zvec = jnp.zeros((_LANES,), jnp.float32)

    def gather(k):
        b = wid * _SPW + k // _NCH
        c = k % _NCH
        slot = k % _NBUF
        return pltpu.make_async_copy(
            x_hbm.at[b, pl.ds(c * _CH, _CH)], bufs[slot], gsems.at[slot]
        )

    def scatter(k):
        b = wid * _SPW + k // _NCH
        c = k % _NCH
        slot = k % _NBUF
        return pltpu.make_async_copy(
            bufs[slot], o_hbm.at[b, pl.ds(c * _CH, _CH)], ssems.at[slot]
        )

    nk = _SPW * _NCH
    for k in range(min(_NBUF - 1, nk)):
        gather(k).start()

    lo = hi = None
    for k in range(nk):
        c = k % _NCH
        if c == 0:
            lo = lo_vec[k // _NCH]
            hi = hi_vec[k // _NCH]
        gather(k).wait()
        c0 = c * _CH
        s = jnp.clip(lo, c0, c0 + _CH) - c0
        e = jnp.clip(hi, c0, c0 + _CH) - c0
        buf = bufs[k % _NBUF]

        def zero_row(r, _, buf=buf):
            for seg in range(_T // _LANES):
                buf[r, pl.ds(seg * _LANES, _LANES)] = zvec
            return 0

        lax.fori_loop(s, e, zero_row, 0)
        scatter(k).start()
        if k >= 1:
            scatter(k - 1).wait()
        if k + _NBUF - 1 < nk:
            gather(k + _NBUF - 1).start()
    scatter(nk - 1).wait()


def kernel(x):
    mask_ratio = 16
    xs = jnp.squeeze(x, axis=1)
    B, F, T = xs.shape
    max_mask = F // mask_ratio
    k = jax.random.key(42)
    k1, k2 = jax.random.split(k)
    if max_mask == 1:
        f_widths = jnp.ones((B,), dtype=jnp.int32)
    else:
        f_widths = jax.random.randint(k1, (B,), 1, max_mask).astype(jnp.int32)
    u = jax.random.uniform(k2, (B,))
    f_low = jnp.floor(u * (F - f_widths).astype(jnp.float32)).astype(jnp.int32)
    f_hi = f_low + f_widths

    run = functools.partial(
        pl.kernel,
        out_type=jax.ShapeDtypeStruct((B, F, T), jnp.float32),
        mesh=plsc.VectorSubcoreMesh(core_axis_name="c", subcore_axis_name="s"),
        compiler_params=pltpu.CompilerParams(use_tc_tiling_on_sc=True),
        scratch_types=[
            pltpu.VMEM((_B + _LANES,), jnp.int32),
            pltpu.VMEM((_B + _LANES,), jnp.int32),
            pltpu.VMEM((_CH, _T), jnp.float32),
            pltpu.VMEM((_CH, _T), jnp.float32),
            pltpu.VMEM((_CH, _T), jnp.float32),
            pltpu.VMEM((_CH, _T), jnp.float32),
            pltpu.SemaphoreType.DMA((_NBUF,)),
            pltpu.SemaphoreType.DMA((_NBUF,)),
        ],
    )(_sc_body)
    out = run(f_low, f_hi, xs)
    return out[:, None, :, :]
